# baseline (speedup 1.0000x reference)
import functools

import jax
import jax.numpy as jnp
from jax import lax
from jax.experimental import pallas as pl
from jax.experimental.pallas import tpu as pltpu
from jax.experimental.pallas import tpu_sc as plsc

B = 16384
D = 10
L = 16
NC = 2
NS = 16
NW = NC * NS
BPW = B // NW
CH = 128
NCH = BPW // CH


def _rsqrt(x):
    i = lax.bitcast_convert_type(x, jnp.int32)
    i = jnp.int32(0x5F3759DF) - lax.shift_right_arithmetic(i, jnp.int32(1))
    y = lax.bitcast_convert_type(i, jnp.float32)
    for _ in range(3):
        y = y * (jnp.float32(1.5) - jnp.float32(0.5) * x * y * y)
    return y


_mesh = plsc.VectorSubcoreMesh(core_axis_name="c", subcore_axis_name="s")


@functools.partial(
    pl.kernel,
    mesh=_mesh,
    out_type=jax.ShapeDtypeStruct((B,), jnp.float32),
    compiler_params=pltpu.CompilerParams(
        needs_layout_passes=False, use_tc_tiling_on_sc=False),
    scratch_types=[
        pltpu.VMEM((BPW,), jnp.int32),
        pltpu.VMEM((BPW,), jnp.int32),
        pltpu.VMEM((BPW, D), jnp.float32),
        pltpu.VMEM((BPW, D), jnp.float32),
        pltpu.VMEM((BPW,), jnp.float32),
        pltpu.SemaphoreType.DMA,
    ],
)
def _cosine_kernel(o_id_hbm, e_id_hbm, mentors_hbm, mentees_hbm, out_hbm,
                   oidx_v, eidx_v, orows_v, erows_v, res_v, sem):
    wid = lax.axis_index("s") * NC + lax.axis_index("c")
    base = wid * BPW

    pltpu.sync_copy(o_id_hbm.at[pl.ds(base, BPW)], oidx_v)
    pltpu.sync_copy(e_id_hbm.at[pl.ds(base, BPW)], eidx_v)

    copies = []
    for c in range(NCH):
        sl = pl.ds(c * CH, CH)
        copies.append(pltpu.async_copy(mentors_hbm.at[oidx_v.at[sl]],
                                       orows_v.at[sl], sem))
        copies.append(pltpu.async_copy(mentees_hbm.at[eidx_v.at[sl]],
                                       erows_v.at[sl], sem))
    for cp in copies:
        cp.wait()

    zero = jnp.zeros((L,), jnp.float32)

    def body(g, _):
        rbase = pl.multiple_of(g * L, L)
        rows = lax.iota(jnp.int32, L) + rbase
        dot = zero
        on2 = zero
        en2 = zero
        for d in range(D):
            cols = jnp.full((L,), d, jnp.int32)
            o = plsc.load_gather(orows_v, [rows, cols])
            e = plsc.load_gather(erows_v, [rows, cols])
            dot = dot + o * e
            on2 = on2 + o * o
            en2 = en2 + e * e
        res_v[pl.ds(rbase, L)] = dot * _rsqrt(on2 * en2)
        return 0

    lax.fori_loop(0, BPW // L, body, 0)

    pltpu.sync_copy(res_v, out_hbm.at[pl.ds(base, BPW)])


def kernel(o_id, e_id, mentors, mentees):
    return _cosine_kernel(o_id.astype(jnp.int32), e_id.astype(jnp.int32),
                          mentors, mentees)
